# baseline (speedup 1.0000x reference)
import functools

import jax
import jax.numpy as jnp
from jax import lax
from jax.experimental import pallas as pl
from jax.experimental.pallas import tpu as pltpu
from jax.experimental.pallas import tpu_sc as plsc

_NV = 4096
_D = 256
_TR = 12288
_NNZ = 196608

_NC = 2
_NS = 16
_NW = _NC * _NS

_GS = 64
_SS = 128
_EPW = _NNZ // _NW
_NG = _EPW // _SS
_RPT = _NV // _NW
_CAP = _EPW
_CH = 256


def _sc_bin_body(rows_hbm, cols_hbm, vals_hbm, bm_hbm, bv_hbm, cnts_hbm,
                 rows_v, cols_v, vals_v, mbuf, slot_v, caux, ssem, cnt_smem):
    cid = lax.axis_index("c")
    sid = lax.axis_index("s")
    w = cid * _NS + sid
    wcap = w * _CAP

    pltpu.sync_copy(rows_hbm.at[w], rows_v)
    pltpu.sync_copy(cols_hbm.at[w], cols_v)
    pltpu.sync_copy(vals_hbm.at[w], vals_v)

    for b in range(_NW):
        cnt_smem[b] = 0
    li = lax.iota(jnp.int32, 16)

    def build(g, c):
        for kb in range(_SS // 16):
            sl = pl.ds(16 * kb, 16)
            rv = rows_v[g, sl]
            cv = cols_v[g, sl]
            mbuf[g, sl] = (cv << 7) | (rv & 127)
            bvec = rv >> 7
            sv = jnp.zeros((16,), jnp.int32)
            for l in range(16):
                b = bvec[l]
                cc = cnt_smem[b]
                cnt_smem[b] = cc + 1
                slot = b * (_NW * _CAP) + wcap + cc
                sv = jnp.where(li == l, jnp.full((16,), slot, jnp.int32), sv)
            slot_v[g, sl] = sv
        return c

    lax.fori_loop(0, _NG, build, 0)

    descs = []
    for g in range(_NG):
        descs.append(pltpu.async_copy(mbuf.at[g], bm_hbm.at[slot_v.at[g]], ssem))
        descs.append(pltpu.async_copy(vals_v.at[g], bv_hbm.at[slot_v.at[g]], ssem))
    for d in descs:
        d.wait()

    cv0 = jnp.zeros((16,), jnp.int32)
    cv1 = jnp.zeros((16,), jnp.int32)
    for b in range(16):
        cv0 = jnp.where(li == b, jnp.full((16,), cnt_smem[b], jnp.int32), cv0)
        cv1 = jnp.where(li == b, jnp.full((16,), cnt_smem[16 + b], jnp.int32), cv1)
    caux[0, pl.ds(0, 16)] = cv0
    caux[0, pl.ds(16, 16)] = cv1
    caux[1, pl.ds(0, 16)] = li * _NW + w
    caux[1, pl.ds(16, 16)] = (li + 16) * _NW + w
    pltpu.async_copy(caux.at[0], cnts_hbm.at[caux.at[1]], ssem).wait()


def _sc_bin(rows3, cols3, vals3):
    k = functools.partial(
        pl.kernel,
        out_type=(
            jax.ShapeDtypeStruct((_NW * _NW * _CAP,), jnp.int32),
            jax.ShapeDtypeStruct((_NW * _NW * _CAP,), jnp.float32),
            jax.ShapeDtypeStruct((_NW * _NW,), jnp.int32),
        ),
        mesh=plsc.VectorSubcoreMesh(core_axis_name="c", subcore_axis_name="s"),
        scratch_types=[
            pltpu.VMEM((_NG, _SS), jnp.int32),
            pltpu.VMEM((_NG, _SS), jnp.int32),
            pltpu.VMEM((_NG, _SS), jnp.float32),
            pltpu.VMEM((_NG, _SS), jnp.int32),
            pltpu.VMEM((_NG, _SS), jnp.int32),
            pltpu.VMEM((2, _NW), jnp.int32),
            pltpu.SemaphoreType.DMA,
            pltpu.SMEM((_NW,), jnp.int32),
        ],
    )(_sc_bin_body)
    return k(rows3, cols3, vals3)


def _sc_acc_body(xr_hbm, bm_hbm, bv_hbm, cnts_hbm, out_hbm,
                 cnt_v, mb, vb, cb, lb, vq, gbuf, acc, gsem, ssem, cnt_smem):
    cid = lax.axis_index("c")
    sid = lax.axis_index("s")
    w = cid * _NS + sid
    li = lax.iota(jnp.int32, 16)

    pltpu.sync_copy(cnts_hbm.at[pl.ds(w * _NW, _NW)], cnt_v)
    c0 = cnt_v[pl.ds(0, 16)]
    c1 = cnt_v[pl.ds(16, 16)]
    for s in range(16):
        cnt_smem[s] = c0[s]
        cnt_smem[16 + s] = c1[s]

    zero16 = jnp.zeros((16,), jnp.float32)

    def zrow(r, c):
        for j in range(_D // 16):
            acc[r, pl.ds(16 * j, 16)] = zero16
        return c

    lax.fori_loop(0, _RPT, zrow, 0)

    wbase = w * _NW * _CAP
    nsub = _CH // _GS

    def s_loop(s, c):
        cnt = cnt_smem[s]
        nch = (cnt + (_CH - 1)) >> 8
        base = wbase + s * _CAP

        def c_loop(ch, c2):
            off = base + ch * _CH
            d1 = pltpu.async_copy(bm_hbm.at[pl.ds(off, _CH)], mb, ssem)
            d2 = pltpu.async_copy(bv_hbm.at[pl.ds(off, _CH)], vb, ssem)
            d1.wait()
            d2.wait()
            rem0 = cnt - ch * _CH

            def prep(q):
                p = q % 2
                remv = jnp.full((16,), rem0 - q * _GS, jnp.int32)
                for j in range(_GS // 16):
                    sl = pl.ds(16 * j, 16)
                    slq = pl.ds(q * _GS + 16 * j, 16)
                    msk = (li + 16 * j) < remv
                    mm = jnp.where(msk, mb[slq], 0)
                    cb[p, sl] = mm >> 7
                    lb[p, sl] = mm & 127
                    vq[p, sl] = jnp.where(msk, vb[slq], 0.0)

            prep(0)
            descs = [pltpu.async_copy(xr_hbm.at[cb.at[0]], gbuf.at[0], gsem)]
            for q in range(nsub):
                p = q % 2
                if q + 1 < nsub:
                    prep(q + 1)
                    descs.append(pltpu.async_copy(
                        xr_hbm.at[cb.at[(q + 1) % 2]],
                        gbuf.at[(q + 1) % 2], gsem))
                descs[q].wait()

                def kb_loop(kb, c3):
                    sl = pl.ds(16 * kb, 16)
                    rv = lb[p, sl]
                    vv = vq[p, sl]
                    for l in range(16):
                        lr = rv[l]
                        val = jnp.full((16,), vv[l], jnp.float32)
                        kk = kb * 16 + l
                        for j in range(_D // 16):
                            sl2 = pl.ds(16 * j, 16)
                            plsc.addupdate(acc.at[lr, sl2],
                                           gbuf[p, kk, sl2] * val)
                    return c3

                lax.fori_loop(0, _GS // 16, kb_loop, 0)
            return c2

        lax.fori_loop(0, nch, c_loop, 0)
        return c

    lax.fori_loop(0, _NW, s_loop, 0)

    pltpu.sync_copy(acc, out_hbm.at[pl.ds(w * _RPT, _RPT)])


def _sc_acc(xr, bm, bv, cnts):
    k = functools.partial(
        pl.kernel,
        out_type=jax.ShapeDtypeStruct((_NV, _D), jnp.float32),
        mesh=plsc.VectorSubcoreMesh(core_axis_name="c", subcore_axis_name="s"),
        scratch_types=[
            pltpu.VMEM((_NW,), jnp.int32),
            pltpu.VMEM((_CH,), jnp.int32),
            pltpu.VMEM((_CH,), jnp.float32),
            pltpu.VMEM((2, _GS), jnp.int32),
            pltpu.VMEM((2, _GS), jnp.int32),
            pltpu.VMEM((2, _GS), jnp.float32),
            pltpu.VMEM((2, _GS, _D), jnp.float32),
            pltpu.VMEM((_RPT, _D), jnp.float32),
            pltpu.SemaphoreType.DMA,
            pltpu.SemaphoreType.DMA,
            pltpu.SMEM((_NW,), jnp.int32),
        ],
    )(_sc_acc_body)
    return k(xr, bm, bv, cnts)


def _mm_body(a_ref, w_ref, b_ref, o_ref):
    o_ref[...] = jnp.dot(a_ref[...], w_ref[...],
                         preferred_element_type=jnp.float32) + b_ref[...]


def _tc_matmul(acc, w2, b2):
    m = acc.shape[0]
    bm = 4096
    return pl.pallas_call(
        _mm_body,
        grid=(m // bm,),
        in_specs=[
            pl.BlockSpec((bm, 32), lambda i: (i, 0)),
            pl.BlockSpec((32, 96), lambda i: (0, 0)),
            pl.BlockSpec((1, 96), lambda i: (0, 0)),
        ],
        out_specs=pl.BlockSpec((bm, 96), lambda i: (i, 0)),
        out_shape=jax.ShapeDtypeStruct((m, 96), jnp.float32),
    )(acc, w2, b2)


def kernel(x, weight, bias, cheb_vals, cheb_rows, cheb_cols):
    xr = x.reshape(_TR, _D)
    rows3 = cheb_rows.reshape(_NW, _NG, _SS)
    cols3 = cheb_cols.reshape(_NW, _NG, _SS)
    vals3 = cheb_vals.reshape(_NW, _NG, _SS)
    bm, bv, cnts = _sc_bin(rows3, cols3, vals3)
    acc = _sc_acc(xr, bm, bv, cnts)
    w2 = weight.reshape(32, 96)
    b2 = jnp.tile(bias, 3).reshape(1, 96)
    out = _tc_matmul(acc.reshape(32768, 32), w2, b2)
    return out.reshape(98304, 32)


# speedup vs baseline: 1.5762x; 1.5762x over previous
import functools

import jax
import jax.numpy as jnp
from jax import lax
from jax.experimental import pallas as pl
from jax.experimental.pallas import tpu as pltpu
from jax.experimental.pallas import tpu_sc as plsc

_NV = 4096
_D = 256
_TR = 12288
_NNZ = 196608

_NC = 2
_NS = 16
_NW = _NC * _NS

_GS = 64
_SS = 128
_EPW = _NNZ // _NW
_NG = _EPW // _SS
_RPT = _NV // _NW
_CAP = _EPW
_CH = 256


def _sc_bin_body(rows_hbm, cols_hbm, vals_hbm, bm_hbm, bv_hbm, cnts_hbm,
                 rows_v, cols_v, vals_v, mbuf, slot_v, caux, ssem, cnt_smem):
    cid = lax.axis_index("c")
    sid = lax.axis_index("s")
    w = cid * _NS + sid
    wcap = w * _CAP

    pltpu.sync_copy(rows_hbm.at[w], rows_v)
    pltpu.sync_copy(cols_hbm.at[w], cols_v)
    pltpu.sync_copy(vals_hbm.at[w], vals_v)

    for b in range(_NW):
        cnt_smem[b] = 0
    li = lax.iota(jnp.int32, 16)

    def build(g, c):
        for kb in range(_SS // 16):
            sl = pl.ds(16 * kb, 16)
            rv = rows_v[g, sl]
            cv = cols_v[g, sl]
            mbuf[g, sl] = (cv << 7) | (rv & 127)
            bvec = rv >> 7
            sv = jnp.zeros((16,), jnp.int32)
            for l in range(16):
                b = bvec[l]
                cc = cnt_smem[b]
                cnt_smem[b] = cc + 1
                slot = b * (_NW * _CAP) + wcap + cc
                sv = jnp.where(li == l, jnp.full((16,), slot, jnp.int32), sv)
            slot_v[g, sl] = sv
        return c

    lax.fori_loop(0, _NG, build, 0)

    descs = []
    for g in range(_NG):
        descs.append(pltpu.async_copy(mbuf.at[g], bm_hbm.at[slot_v.at[g]], ssem))
        descs.append(pltpu.async_copy(vals_v.at[g], bv_hbm.at[slot_v.at[g]], ssem))
    for d in descs:
        d.wait()

    cv0 = jnp.zeros((16,), jnp.int32)
    cv1 = jnp.zeros((16,), jnp.int32)
    for b in range(16):
        cv0 = jnp.where(li == b, jnp.full((16,), cnt_smem[b], jnp.int32), cv0)
        cv1 = jnp.where(li == b, jnp.full((16,), cnt_smem[16 + b], jnp.int32), cv1)
    caux[0, pl.ds(0, 16)] = cv0
    caux[0, pl.ds(16, 16)] = cv1
    caux[1, pl.ds(0, 16)] = li * _NW + w
    caux[1, pl.ds(16, 16)] = (li + 16) * _NW + w
    pltpu.async_copy(caux.at[0], cnts_hbm.at[caux.at[1]], ssem).wait()


def _sc_bin(rows3, cols3, vals3):
    k = functools.partial(
        pl.kernel,
        out_type=(
            jax.ShapeDtypeStruct((_NW * _NW * _CAP,), jnp.int32),
            jax.ShapeDtypeStruct((_NW * _NW * _CAP,), jnp.float32),
            jax.ShapeDtypeStruct((_NW * _NW,), jnp.int32),
        ),
        mesh=plsc.VectorSubcoreMesh(core_axis_name="c", subcore_axis_name="s"),
        scratch_types=[
            pltpu.VMEM((_NG, _SS), jnp.int32),
            pltpu.VMEM((_NG, _SS), jnp.int32),
            pltpu.VMEM((_NG, _SS), jnp.float32),
            pltpu.VMEM((_NG, _SS), jnp.int32),
            pltpu.VMEM((_NG, _SS), jnp.int32),
            pltpu.VMEM((2, _NW), jnp.int32),
            pltpu.SemaphoreType.DMA,
            pltpu.SMEM((_NW,), jnp.int32),
        ],
    )(_sc_bin_body)
    return k(rows3, cols3, vals3)


def _sc_acc_body(xr_hbm, bm_hbm, bv_hbm, cnts_hbm, out_hbm,
                 cnt_v, mb, vb, cb, lb, vq, gbuf, acc, gsem, ssem, cnt_smem):
    cid = lax.axis_index("c")
    sid = lax.axis_index("s")
    w = cid * _NS + sid
    li = lax.iota(jnp.int32, 16)

    pltpu.sync_copy(cnts_hbm.at[pl.ds(w * _NW, _NW)], cnt_v)
    c0 = cnt_v[pl.ds(0, 16)]
    c1 = cnt_v[pl.ds(16, 16)]
    for s in range(16):
        cnt_smem[s] = c0[s]
        cnt_smem[16 + s] = c1[s]

    zero16 = jnp.zeros((16,), jnp.float32)

    def zrow(r, c):
        for j in range(_D // 16):
            acc[r, pl.ds(16 * j, 16)] = zero16
        return c

    lax.fori_loop(0, _RPT, zrow, 0)

    wbase = w * _NW * _CAP
    nsub = _CH // _GS

    def s_loop(s, c):
        cnt = cnt_smem[s]
        nch = (cnt + (_CH - 1)) >> 8
        base = wbase + s * _CAP

        def c_loop(ch, c2):
            off = base + ch * _CH
            d1 = pltpu.async_copy(bm_hbm.at[pl.ds(off, _CH)], mb, ssem)
            d2 = pltpu.async_copy(bv_hbm.at[pl.ds(off, _CH)], vb, ssem)
            d1.wait()
            d2.wait()
            rem0 = cnt - ch * _CH
            nq = (jnp.minimum(rem0, _CH) + (_GS - 1)) >> 6

            def q_loop(q, c3):
                qoff = q * _GS
                remv = jnp.full((16,), rem0 - qoff, jnp.int32)
                for j in range(_GS // 16):
                    sl = pl.ds(16 * j, 16)
                    slq = pl.ds(qoff + 16 * j, 16)
                    msk = (li + 16 * j) < remv
                    mm = jnp.where(msk, mb[slq], 0)
                    cb[0, sl] = mm >> 7
                    lb[0, sl] = mm & 127
                    vq[0, sl] = jnp.where(msk, vb[slq], 0.0)
                pltpu.async_copy(xr_hbm.at[cb.at[0]], gbuf.at[0], gsem).wait()
                for kb in range(_GS // 16):
                    sl = pl.ds(16 * kb, 16)
                    rv = lb[0, sl]
                    vv = vq[0, sl]
                    for l in range(16):
                        lr = rv[l]
                        val = jnp.full((16,), vv[l], jnp.float32)
                        kk = kb * 16 + l
                        for j in range(_D // 16):
                            sl2 = pl.ds(16 * j, 16)
                            plsc.addupdate(acc.at[lr, sl2],
                                           gbuf[0, kk, sl2] * val)
                return c3

            lax.fori_loop(0, nq, q_loop, 0)
            return c2

        lax.fori_loop(0, nch, c_loop, 0)
        return c

    lax.fori_loop(0, _NW, s_loop, 0)

    pltpu.sync_copy(acc, out_hbm.at[pl.ds(w * _RPT, _RPT)])


def _sc_acc(xr, bm, bv, cnts):
    k = functools.partial(
        pl.kernel,
        out_type=jax.ShapeDtypeStruct((_NV, _D), jnp.float32),
        mesh=plsc.VectorSubcoreMesh(core_axis_name="c", subcore_axis_name="s"),
        scratch_types=[
            pltpu.VMEM((_NW,), jnp.int32),
            pltpu.VMEM((_CH,), jnp.int32),
            pltpu.VMEM((_CH,), jnp.float32),
            pltpu.VMEM((2, _GS), jnp.int32),
            pltpu.VMEM((2, _GS), jnp.int32),
            pltpu.VMEM((2, _GS), jnp.float32),
            pltpu.VMEM((2, _GS, _D), jnp.float32),
            pltpu.VMEM((_RPT, _D), jnp.float32),
            pltpu.SemaphoreType.DMA,
            pltpu.SemaphoreType.DMA,
            pltpu.SMEM((_NW,), jnp.int32),
        ],
    )(_sc_acc_body)
    return k(xr, bm, bv, cnts)


def _mm_body(a_ref, w_ref, b_ref, o_ref):
    o_ref[...] = jnp.dot(a_ref[...], w_ref[...],
                         preferred_element_type=jnp.float32) + b_ref[...]


def _tc_matmul(acc, w2, b2):
    m = acc.shape[0]
    bm = 4096
    return pl.pallas_call(
        _mm_body,
        grid=(m // bm,),
        in_specs=[
            pl.BlockSpec((bm, 32), lambda i: (i, 0)),
            pl.BlockSpec((32, 96), lambda i: (0, 0)),
            pl.BlockSpec((1, 96), lambda i: (0, 0)),
        ],
        out_specs=pl.BlockSpec((bm, 96), lambda i: (i, 0)),
        out_shape=jax.ShapeDtypeStruct((m, 96), jnp.float32),
    )(acc, w2, b2)


def kernel(x, weight, bias, cheb_vals, cheb_rows, cheb_cols):
    xr = x.reshape(_TR, _D)
    rows3 = cheb_rows.reshape(_NW, _NG, _SS)
    cols3 = cheb_cols.reshape(_NW, _NG, _SS)
    vals3 = cheb_vals.reshape(_NW, _NG, _SS)
    bm, bv, cnts = _sc_bin(rows3, cols3, vals3)
    acc = _sc_acc(xr, bm, bv, cnts)
    w2 = weight.reshape(32, 96)
    b2 = jnp.tile(bias, 3).reshape(1, 96)
    out = _tc_matmul(acc.reshape(32768, 32), w2, b2)
    return out.reshape(98304, 32)


# speedup vs baseline: 1.6238x; 1.0302x over previous
import functools

import jax
import jax.numpy as jnp
from jax import lax
from jax.experimental import pallas as pl
from jax.experimental.pallas import tpu as pltpu
from jax.experimental.pallas import tpu_sc as plsc

_NV = 4096
_D = 256
_TR = 12288
_NNZ = 196608

_NC = 2
_NS = 16
_NW = _NC * _NS

_GS = 64
_SS = 128
_EPW = _NNZ // _NW
_NG = _EPW // _SS
_RPT = _NV // _NW
_CAP = _EPW
_CH = 256


def _sc_bin_body(rows_hbm, cols_hbm, vals_hbm, bm_hbm, bv_hbm, cnts_hbm,
                 rows_v, cols_v, vals_v, mbuf, slot_v, caux, ssem, cnt_smem):
    cid = lax.axis_index("c")
    sid = lax.axis_index("s")
    w = cid * _NS + sid
    wcap = w * _CAP

    pltpu.sync_copy(rows_hbm.at[w], rows_v)
    pltpu.sync_copy(cols_hbm.at[w], cols_v)
    pltpu.sync_copy(vals_hbm.at[w], vals_v)

    for b in range(_NW):
        cnt_smem[b] = 0
    li = lax.iota(jnp.int32, 16)

    def build(g, c):
        for kb in range(_SS // 16):
            sl = pl.ds(16 * kb, 16)
            rv = rows_v[g, sl]
            cv = cols_v[g, sl]
            mbuf[g, sl] = (cv << 7) | (rv & 127)
            bvec = rv >> 7
            sv = jnp.zeros((16,), jnp.int32)
            for l in range(16):
                b = bvec[l]
                cc = cnt_smem[b]
                cnt_smem[b] = cc + 1
                slot = b * (_NW * _CAP) + wcap + cc
                sv = jnp.where(li == l, jnp.full((16,), slot, jnp.int32), sv)
            slot_v[g, sl] = sv
        return c

    lax.fori_loop(0, _NG, build, 0)

    descs = []
    for g in range(_NG):
        descs.append(pltpu.async_copy(mbuf.at[g], bm_hbm.at[slot_v.at[g]], ssem))
        descs.append(pltpu.async_copy(vals_v.at[g], bv_hbm.at[slot_v.at[g]], ssem))
    for d in descs:
        d.wait()

    cv0 = jnp.zeros((16,), jnp.int32)
    cv1 = jnp.zeros((16,), jnp.int32)
    for b in range(16):
        cv0 = jnp.where(li == b, jnp.full((16,), cnt_smem[b], jnp.int32), cv0)
        cv1 = jnp.where(li == b, jnp.full((16,), cnt_smem[16 + b], jnp.int32), cv1)
    caux[0, pl.ds(0, 16)] = cv0
    caux[0, pl.ds(16, 16)] = cv1
    caux[1, pl.ds(0, 16)] = li * _NW + w
    caux[1, pl.ds(16, 16)] = (li + 16) * _NW + w
    pltpu.async_copy(caux.at[0], cnts_hbm.at[caux.at[1]], ssem).wait()


def _sc_bin(rows3, cols3, vals3):
    k = functools.partial(
        pl.kernel,
        out_type=(
            jax.ShapeDtypeStruct((_NW * _NW * _CAP,), jnp.int32),
            jax.ShapeDtypeStruct((_NW * _NW * _CAP,), jnp.float32),
            jax.ShapeDtypeStruct((_NW * _NW,), jnp.int32),
        ),
        mesh=plsc.VectorSubcoreMesh(core_axis_name="c", subcore_axis_name="s"),
        scratch_types=[
            pltpu.VMEM((_NG, _SS), jnp.int32),
            pltpu.VMEM((_NG, _SS), jnp.int32),
            pltpu.VMEM((_NG, _SS), jnp.float32),
            pltpu.VMEM((_NG, _SS), jnp.int32),
            pltpu.VMEM((_NG, _SS), jnp.int32),
            pltpu.VMEM((2, _NW), jnp.int32),
            pltpu.SemaphoreType.DMA,
            pltpu.SMEM((_NW,), jnp.int32),
        ],
    )(_sc_bin_body)
    return k(rows3, cols3, vals3)


def _sc_acc_body(xr_hbm, bm_hbm, bv_hbm, cnts_hbm, out_hbm,
                 cnt_v, mb, vb, cb, lb, vq, gbuf, acc, gsem, ssem, cnt_smem):
    cid = lax.axis_index("c")
    sid = lax.axis_index("s")
    w = cid * _NS + sid
    li = lax.iota(jnp.int32, 16)

    pltpu.sync_copy(cnts_hbm.at[pl.ds(w * _NW, _NW)], cnt_v)
    c0 = cnt_v[pl.ds(0, 16)]
    c1 = cnt_v[pl.ds(16, 16)]
    for s in range(16):
        cnt_smem[s] = c0[s]
        cnt_smem[16 + s] = c1[s]

    zero16 = jnp.zeros((16,), jnp.float32)

    def zrow(r, c):
        for j in range(_D // 16):
            acc[r, pl.ds(16 * j, 16)] = zero16
        return c

    lax.fori_loop(0, _RPT, zrow, 0)

    wbase = w * _NW * _CAP
    nsub = _CH // _GS

    def s_loop(s, c):
        cnt = cnt_smem[s]
        nch = (cnt + (_CH - 1)) >> 8
        base = wbase + s * _CAP

        def c_loop(ch, c2):
            off = base + ch * _CH
            d1 = pltpu.async_copy(bm_hbm.at[pl.ds(off, _CH)], mb, ssem)
            d2 = pltpu.async_copy(bv_hbm.at[pl.ds(off, _CH)], vb, ssem)
            d1.wait()
            d2.wait()
            rem0 = cnt - ch * _CH
            nq = (jnp.minimum(rem0, _CH) + (_GS - 1)) >> 6

            def prep_fire(q, p):
                qoff = q * _GS
                remv = jnp.full((16,), rem0 - qoff, jnp.int32)
                for j in range(_GS // 16):
                    sl = pl.ds(16 * j, 16)
                    slq = pl.ds(qoff + 16 * j, 16)
                    msk = (li + 16 * j) < remv
                    mm = jnp.where(msk, mb[slq], 0)
                    cb[p, sl] = mm >> 7
                    lb[p, sl] = mm & 127
                    vq[p, sl] = jnp.where(msk, vb[slq], 0.0)
                pltpu.async_copy(xr_hbm.at[cb.at[p]], gbuf.at[p], gsem)

            prep_fire(0, 0)

            def q_loop(q, c3):
                p = q & 1
                pltpu.make_async_copy(xr_hbm.at[cb.at[p]],
                                      gbuf.at[p], gsem).wait()

                @pl.when(q + 1 < nq)
                def _():
                    prep_fire(q + 1, 1 - p)

                for kb in range(_GS // 16):
                    sl = pl.ds(16 * kb, 16)
                    rv = lb[p, sl]
                    vv = vq[p, sl]
                    for l in range(16):
                        lr = rv[l]
                        val = jnp.full((16,), vv[l], jnp.float32)
                        kk = kb * 16 + l
                        for j in range(_D // 16):
                            sl2 = pl.ds(16 * j, 16)
                            plsc.addupdate(acc.at[lr, sl2],
                                           gbuf[p, kk, sl2] * val)
                return c3

            lax.fori_loop(0, nq, q_loop, 0)
            return c2

        lax.fori_loop(0, nch, c_loop, 0)
        return c

    lax.fori_loop(0, _NW, s_loop, 0)

    pltpu.sync_copy(acc, out_hbm.at[pl.ds(w * _RPT, _RPT)])


def _sc_acc(xr, bm, bv, cnts):
    k = functools.partial(
        pl.kernel,
        out_type=jax.ShapeDtypeStruct((_NV, _D), jnp.float32),
        mesh=plsc.VectorSubcoreMesh(core_axis_name="c", subcore_axis_name="s"),
        scratch_types=[
            pltpu.VMEM((_NW,), jnp.int32),
            pltpu.VMEM((_CH,), jnp.int32),
            pltpu.VMEM((_CH,), jnp.float32),
            pltpu.VMEM((2, _GS), jnp.int32),
            pltpu.VMEM((2, _GS), jnp.int32),
            pltpu.VMEM((2, _GS), jnp.float32),
            pltpu.VMEM((2, _GS, _D), jnp.float32),
            pltpu.VMEM((_RPT, _D), jnp.float32),
            pltpu.SemaphoreType.DMA,
            pltpu.SemaphoreType.DMA,
            pltpu.SMEM((_NW,), jnp.int32),
        ],
    )(_sc_acc_body)
    return k(xr, bm, bv, cnts)


def _mm_body(a_ref, w_ref, b_ref, o_ref):
    o_ref[...] = jnp.dot(a_ref[...], w_ref[...],
                         preferred_element_type=jnp.float32) + b_ref[...]


def _tc_matmul(acc, w2, b2):
    m = acc.shape[0]
    bm = 4096
    return pl.pallas_call(
        _mm_body,
        grid=(m // bm,),
        in_specs=[
            pl.BlockSpec((bm, 32), lambda i: (i, 0)),
            pl.BlockSpec((32, 96), lambda i: (0, 0)),
            pl.BlockSpec((1, 96), lambda i: (0, 0)),
        ],
        out_specs=pl.BlockSpec((bm, 96), lambda i: (i, 0)),
        out_shape=jax.ShapeDtypeStruct((m, 96), jnp.float32),
    )(acc, w2, b2)


def kernel(x, weight, bias, cheb_vals, cheb_rows, cheb_cols):
    xr = x.reshape(_TR, _D)
    rows3 = cheb_rows.reshape(_NW, _NG, _SS)
    cols3 = cheb_cols.reshape(_NW, _NG, _SS)
    vals3 = cheb_vals.reshape(_NW, _NG, _SS)
    bm, bv, cnts = _sc_bin(rows3, cols3, vals3)
    acc = _sc_acc(xr, bm, bv, cnts)
    w2 = weight.reshape(32, 96)
    b2 = jnp.tile(bias, 3).reshape(1, 96)
    out = _tc_matmul(acc.reshape(32768, 32), w2, b2)
    return out.reshape(98304, 32)
